# Initial kernel scaffold; baseline (speedup 1.0000x reference)
import functools

import jax
import jax.numpy as jnp
from jax import lax
from jax.experimental import pallas as pl
from jax.experimental.pallas import tpu as pltpu
from jax.experimental.pallas import tpu_sc as plsc

N = 10000
E = 320000
F = 128
U = 128

NC = 2
NS = 16
LANES = 16

E_PER_TILE = E // (NC * NS)
WIN = 80
NWIN = E_PER_TILE // WIN
ROWS_PER_TILE = N // NS
NPAD = 10240
DEG_PER_TILE = NPAD // NS

_mesh = plsc.VectorSubcoreMesh(core_axis_name="c", subcore_axis_name="s")


@functools.partial(
    pl.kernel,
    mesh=_mesh,
    out_type=jax.ShapeDtypeStruct((NC, NPAD), jnp.float32),
    scratch_types=[
        pltpu.VMEM((WIN,), jnp.int32),
        pltpu.VMEM((WIN,), jnp.float32),
        pltpu.VMEM((DEG_PER_TILE,), jnp.float32),
        pltpu.VMEM_SHARED((NPAD,), jnp.float32),
    ],
)
def _deg_kernel(row_hbm, out_hbm, idx_v, ones_v, z_v, hist_sh):
    cid = lax.axis_index("c")
    sid = lax.axis_index("s")

    one = jnp.ones((LANES,), jnp.float32)
    zero = jnp.zeros((LANES,), jnp.float32)
    for j in range(WIN // LANES):
        ones_v[pl.ds(j * LANES, LANES)] = one

    def _zwrite(i, carry):
        z_v[pl.ds(i * LANES, LANES)] = zero
        return carry

    lax.fori_loop(0, DEG_PER_TILE // LANES, _zwrite, 0)
    pltpu.sync_copy(z_v, hist_sh.at[pl.ds(sid * DEG_PER_TILE, DEG_PER_TILE)])
    plsc.subcore_barrier()

    tile_base = (cid * NS + sid) * E_PER_TILE

    def _win(w, carry):
        pltpu.sync_copy(row_hbm.at[pl.ds(tile_base + w * WIN, WIN)], idx_v)
        pltpu.sync_copy(ones_v, hist_sh.at[idx_v], add=True)
        return carry

    lax.fori_loop(0, NWIN, _win, 0)
    plsc.subcore_barrier()

    pltpu.sync_copy(
        hist_sh.at[pl.ds(sid * DEG_PER_TILE, DEG_PER_TILE)],
        out_hbm.at[cid, pl.ds(sid * DEG_PER_TILE, DEG_PER_TILE)],
    )


@functools.partial(
    pl.kernel,
    mesh=_mesh,
    out_type=jax.ShapeDtypeStruct((NC, N, U), jnp.float32),
    scratch_types=[
        pltpu.VMEM((WIN,), jnp.int32),
        pltpu.VMEM((WIN,), jnp.int32),
        pltpu.VMEM((WIN, U), jnp.float32),
        pltpu.VMEM_SHARED((N, U), jnp.float32),
        pltpu.SemaphoreType.DMA,
    ],
)
def _prop_kernel(h2_hbm, row_hbm, col_hbm, out_hbm, idxr_v, idxc_v, g_v,
                 acc_sh, sem):
    cid = lax.axis_index("c")
    sid = lax.axis_index("s")

    rbase = sid * ROWS_PER_TILE
    pltpu.sync_copy(h2_hbm.at[pl.ds(rbase, ROWS_PER_TILE)],
                    acc_sh.at[pl.ds(rbase, ROWS_PER_TILE)])
    plsc.subcore_barrier()

    tile_base = (cid * NS + sid) * E_PER_TILE

    def _win(w, carry):
        base = tile_base + w * WIN
        pltpu.sync_copy(row_hbm.at[pl.ds(base, WIN)], idxr_v)
        pltpu.sync_copy(col_hbm.at[pl.ds(base, WIN)], idxc_v)
        pltpu.async_copy(h2_hbm.at[idxr_v], g_v, sem).wait()
        pltpu.sync_copy(g_v, acc_sh.at[idxc_v], add=True)
        return carry

    lax.fori_loop(0, NWIN, _win, 0)
    plsc.subcore_barrier()

    pltpu.sync_copy(acc_sh.at[pl.ds(rbase, ROWS_PER_TILE)],
                    out_hbm.at[cid, pl.ds(rbase, ROWS_PER_TILE)])


def _h2_body(x_ref, w_ref, d0_ref, d1_ref, h2_ref):
    deg = d0_ref[...] + d1_ref[...] + 1.0
    dinv = lax.rsqrt(deg)
    h = jnp.dot(x_ref[...], w_ref[...], preferred_element_type=jnp.float32)
    h2_ref[...] = h * dinv


def _combine_body(p0_ref, p1_ref, h2_ref, d0_ref, d1_ref, b_ref, o_ref):
    deg = d0_ref[...] + d1_ref[...] + 1.0
    dinv = lax.rsqrt(deg)
    s = p0_ref[0] + p1_ref[0] - h2_ref[...]
    o_ref[...] = s * dinv + b_ref[...]


_BLK = 2000


def kernel(x, edge_index, kernel, bias):
    row = edge_index[0]
    col = edge_index[1]

    deg_part = _deg_kernel(row)
    d0 = deg_part[0, :N].reshape(N, 1)
    d1 = deg_part[1, :N].reshape(N, 1)

    grid = N // _BLK
    h2 = pl.pallas_call(
        _h2_body,
        grid=(grid,),
        in_specs=[
            pl.BlockSpec((_BLK, F), lambda i: (i, 0)),
            pl.BlockSpec((F, U), lambda i: (0, 0)),
            pl.BlockSpec((_BLK, 1), lambda i: (i, 0)),
            pl.BlockSpec((_BLK, 1), lambda i: (i, 0)),
        ],
        out_specs=pl.BlockSpec((_BLK, U), lambda i: (i, 0)),
        out_shape=jax.ShapeDtypeStruct((N, U), jnp.float32),
    )(x, kernel, d0, d1)

    p = _prop_kernel(h2, row, col)

    out = pl.pallas_call(
        _combine_body,
        grid=(grid,),
        in_specs=[
            pl.BlockSpec((1, _BLK, U), lambda i: (0, i, 0)),
            pl.BlockSpec((1, _BLK, U), lambda i: (1, i, 0)),
            pl.BlockSpec((_BLK, U), lambda i: (i, 0)),
            pl.BlockSpec((_BLK, 1), lambda i: (i, 0)),
            pl.BlockSpec((_BLK, 1), lambda i: (i, 0)),
            pl.BlockSpec((1, U), lambda i: (0, 0)),
        ],
        out_specs=pl.BlockSpec((_BLK, U), lambda i: (i, 0)),
        out_shape=jax.ShapeDtypeStruct((N, U), jnp.float32),
    )(p, p, h2, d0, d1, bias.reshape(1, U))
    return out


# speedup vs baseline: 17.8985x; 17.8985x over previous
import functools

import jax
import jax.numpy as jnp
from jax import lax
from jax.experimental import pallas as pl
from jax.experimental.pallas import tpu as pltpu
from jax.experimental.pallas import tpu_sc as plsc

N = 10000
E = 320000
F = 128
U = 128

NC = 2
NS = 16
LANES = 16

E_PER_TILE = E // (NC * NS)
WIN = 80
NWIN = E_PER_TILE // WIN
ROWS_A = 624
ROWS_B = N - 15 * ROWS_A
NPAD = 10240
DEG_PER_TILE = NPAD // NS

_mesh = plsc.VectorSubcoreMesh(core_axis_name="c", subcore_axis_name="s")


@functools.partial(
    pl.kernel,
    mesh=_mesh,
    out_type=jax.ShapeDtypeStruct((NC * NPAD,), jnp.float32),
    scratch_types=[
        pltpu.VMEM((WIN,), jnp.int32),
        pltpu.VMEM((WIN,), jnp.float32),
        pltpu.VMEM((DEG_PER_TILE,), jnp.float32),
        pltpu.VMEM_SHARED((NPAD,), jnp.float32),
    ],
)
def _deg_kernel(row_hbm, out_hbm, idx_v, ones_v, z_v, hist_sh):
    cid = lax.axis_index("c")
    sid = lax.axis_index("s")

    one = jnp.ones((LANES,), jnp.float32)
    zero = jnp.zeros((LANES,), jnp.float32)
    for j in range(WIN // LANES):
        ones_v[pl.ds(j * LANES, LANES)] = one

    def _zwrite(i, carry):
        z_v[pl.ds(i * LANES, LANES)] = zero
        return carry

    lax.fori_loop(0, DEG_PER_TILE // LANES, _zwrite, 0)
    pltpu.sync_copy(z_v, hist_sh.at[pl.ds(sid * DEG_PER_TILE, DEG_PER_TILE)])
    plsc.subcore_barrier()

    tile_base = (cid * NS + sid) * E_PER_TILE

    def _win(w, carry):
        pltpu.sync_copy(row_hbm.at[pl.ds(tile_base + w * WIN, WIN)], idx_v)
        pltpu.sync_copy(ones_v, hist_sh.at[idx_v], add=True)
        return carry

    lax.fori_loop(0, NWIN, _win, 0)
    plsc.subcore_barrier()

    pltpu.sync_copy(
        hist_sh.at[pl.ds(sid * DEG_PER_TILE, DEG_PER_TILE)],
        out_hbm.at[pl.ds(cid * NPAD + sid * DEG_PER_TILE, DEG_PER_TILE)],
    )


@functools.partial(
    pl.kernel,
    mesh=_mesh,
    out_type=jax.ShapeDtypeStruct((NC, N, U), jnp.float32),
    scratch_types=[
        pltpu.VMEM((WIN,), jnp.int32),
        pltpu.VMEM((WIN,), jnp.int32),
        pltpu.VMEM((WIN, U), jnp.float32),
        pltpu.VMEM_SHARED((N, U), jnp.float32),
        pltpu.SemaphoreType.DMA,
    ],
)
def _prop_kernel(h2_hbm, row_hbm, col_hbm, out_hbm, idxr_v, idxc_v, g_v,
                 acc_sh, sem):
    cid = lax.axis_index("c")
    sid = lax.axis_index("s")

    rbase = sid * ROWS_A

    @pl.when(sid < NS - 1)
    def _():
        pltpu.sync_copy(h2_hbm.at[pl.ds(rbase, ROWS_A)],
                        acc_sh.at[pl.ds(rbase, ROWS_A)])

    @pl.when(sid == NS - 1)
    def _():
        pltpu.sync_copy(h2_hbm.at[pl.ds(15 * ROWS_A, ROWS_B)],
                        acc_sh.at[pl.ds(15 * ROWS_A, ROWS_B)])

    plsc.subcore_barrier()

    tile_base = (cid * NS + sid) * E_PER_TILE

    def _win(w, carry):
        base = tile_base + w * WIN
        pltpu.sync_copy(row_hbm.at[pl.ds(base, WIN)], idxr_v)
        pltpu.sync_copy(col_hbm.at[pl.ds(base, WIN)], idxc_v)
        pltpu.async_copy(h2_hbm.at[idxr_v], g_v, sem).wait()
        pltpu.sync_copy(g_v, acc_sh.at[idxc_v], add=True)
        return carry

    lax.fori_loop(0, NWIN, _win, 0)
    plsc.subcore_barrier()

    @pl.when(sid < NS - 1)
    def _():
        pltpu.sync_copy(acc_sh.at[pl.ds(rbase, ROWS_A)],
                        out_hbm.at[cid, pl.ds(rbase, ROWS_A)])

    @pl.when(sid == NS - 1)
    def _():
        pltpu.sync_copy(acc_sh.at[pl.ds(15 * ROWS_A, ROWS_B)],
                        out_hbm.at[cid, pl.ds(15 * ROWS_A, ROWS_B)])


def _h2_body(x_ref, w_ref, d0_ref, d1_ref, h2_ref):
    deg = d0_ref[...] + d1_ref[...] + 1.0
    dinv = lax.rsqrt(deg)
    h = jnp.dot(x_ref[...], w_ref[...], preferred_element_type=jnp.float32)
    h2_ref[...] = h * dinv


def _combine_body(p0_ref, p1_ref, h2_ref, d0_ref, d1_ref, b_ref, o_ref):
    deg = d0_ref[...] + d1_ref[...] + 1.0
    dinv = lax.rsqrt(deg)
    s = p0_ref[0] + p1_ref[0] - h2_ref[...]
    o_ref[...] = s * dinv + b_ref[...]


_BLK = 2000


def kernel(x, edge_index, kernel, bias):
    row = edge_index[0]
    col = edge_index[1]

    deg_part = _deg_kernel(row)
    d0 = deg_part[:N].reshape(N, 1)
    d1 = deg_part[NPAD:NPAD + N].reshape(N, 1)

    grid = N // _BLK
    h2 = pl.pallas_call(
        _h2_body,
        grid=(grid,),
        in_specs=[
            pl.BlockSpec((_BLK, F), lambda i: (i, 0)),
            pl.BlockSpec((F, U), lambda i: (0, 0)),
            pl.BlockSpec((_BLK, 1), lambda i: (i, 0)),
            pl.BlockSpec((_BLK, 1), lambda i: (i, 0)),
        ],
        out_specs=pl.BlockSpec((_BLK, U), lambda i: (i, 0)),
        out_shape=jax.ShapeDtypeStruct((N, U), jnp.float32),
    )(x, kernel, d0, d1)

    p = _prop_kernel(h2, row, col)

    out = pl.pallas_call(
        _combine_body,
        grid=(grid,),
        in_specs=[
            pl.BlockSpec((1, _BLK, U), lambda i: (0, i, 0)),
            pl.BlockSpec((1, _BLK, U), lambda i: (1, i, 0)),
            pl.BlockSpec((_BLK, U), lambda i: (i, 0)),
            pl.BlockSpec((_BLK, 1), lambda i: (i, 0)),
            pl.BlockSpec((_BLK, 1), lambda i: (i, 0)),
            pl.BlockSpec((1, U), lambda i: (0, 0)),
        ],
        out_specs=pl.BlockSpec((_BLK, U), lambda i: (i, 0)),
        out_shape=jax.ShapeDtypeStruct((N, U), jnp.float32),
    )(p, p, h2, d0, d1, bias.reshape(1, U))
    return out


# speedup vs baseline: 48.1882x; 2.6923x over previous
import functools

import jax
import jax.numpy as jnp
from jax import lax
from jax.experimental import pallas as pl
from jax.experimental.pallas import tpu as pltpu
from jax.experimental.pallas import tpu_sc as plsc

N = 10000
E = 320000
F = 128
U = 128

NC = 2
NS = 16
LANES = 16

E_PER_TILE = E // (NC * NS)
WIN = 80
NWIN = E_PER_TILE // WIN
ROWS_A = 624
ROWS_B = N - 15 * ROWS_A
NPAD = 10240
DEG_PER_TILE = NPAD // NS

_mesh = plsc.VectorSubcoreMesh(core_axis_name="c", subcore_axis_name="s")


@functools.partial(
    pl.kernel,
    mesh=_mesh,
    out_type=jax.ShapeDtypeStruct((NC * NPAD,), jnp.float32),
    scratch_types=[
        pltpu.VMEM((NWIN, WIN), jnp.int32),
        pltpu.VMEM((WIN,), jnp.float32),
        pltpu.VMEM((DEG_PER_TILE,), jnp.float32),
        pltpu.VMEM_SHARED((NPAD,), jnp.float32),
        pltpu.SemaphoreType.DMA,
    ],
)
def _deg_kernel(row3_hbm, out_hbm, idx_v, ones_v, z_v, hist_sh, sem):
    cid = lax.axis_index("c")
    sid = lax.axis_index("s")
    tid = cid * NS + sid

    one = jnp.ones((LANES,), jnp.float32)
    zero = jnp.zeros((LANES,), jnp.float32)
    for j in range(WIN // LANES):
        ones_v[pl.ds(j * LANES, LANES)] = one

    def _zwrite(i, carry):
        z_v[pl.ds(i * LANES, LANES)] = zero
        return carry

    lax.fori_loop(0, DEG_PER_TILE // LANES, _zwrite, 0)
    pltpu.sync_copy(z_v, hist_sh.at[pl.ds(sid * DEG_PER_TILE, DEG_PER_TILE)])
    pltpu.sync_copy(row3_hbm.at[tid], idx_v)
    plsc.subcore_barrier()

    def _win(w, carry):
        pltpu.async_copy(ones_v, hist_sh.at[idx_v.at[w]], sem, add=True)
        return carry

    lax.fori_loop(0, NWIN, _win, 0)

    def _drain(w, carry):
        pltpu.make_async_copy(ones_v, hist_sh.at[idx_v.at[0]], sem).wait()
        return carry

    lax.fori_loop(0, NWIN, _drain, 0)
    plsc.subcore_barrier()

    pltpu.sync_copy(
        hist_sh.at[pl.ds(sid * DEG_PER_TILE, DEG_PER_TILE)],
        out_hbm.at[pl.ds(cid * NPAD + sid * DEG_PER_TILE, DEG_PER_TILE)],
    )


GRP = 5
PWIN = 40
PNWIN = E_PER_TILE // PWIN
NGRP = PNWIN // GRP


@functools.partial(
    pl.kernel,
    mesh=_mesh,
    out_type=jax.ShapeDtypeStruct((NC, N, U), jnp.float32),
    scratch_types=[
        pltpu.VMEM((E_PER_TILE,), jnp.int32),
        pltpu.VMEM((GRP, PWIN), jnp.int32),
        pltpu.VMEM((GRP, PWIN, U), jnp.float32),
        pltpu.VMEM_SHARED((N, U), jnp.float32),
        pltpu.SemaphoreType.DMA,
        pltpu.SemaphoreType.DMA,
    ],
)
def _prop_kernel(h2_hbm, row_hbm, col3_hbm, out_hbm, idxr_v, idxc_v, g_v,
                 acc_sh, sem, sem_c):
    cid = lax.axis_index("c")
    sid = lax.axis_index("s")
    tid = cid * NS + sid

    rbase = sid * ROWS_A

    @pl.when(sid < NS - 1)
    def _():
        pltpu.sync_copy(h2_hbm.at[pl.ds(rbase, ROWS_A)],
                        acc_sh.at[pl.ds(rbase, ROWS_A)])

    @pl.when(sid == NS - 1)
    def _():
        pltpu.sync_copy(h2_hbm.at[pl.ds(15 * ROWS_A, ROWS_B)],
                        acc_sh.at[pl.ds(15 * ROWS_A, ROWS_B)])

    pltpu.sync_copy(row_hbm.at[pl.ds(tid * E_PER_TILE, E_PER_TILE)], idxr_v)
    plsc.subcore_barrier()

    for b in range(GRP):
        pltpu.async_copy(col3_hbm.at[tid, b], idxc_v.at[b], sem_c)
        pltpu.async_copy(h2_hbm.at[idxr_v.at[pl.ds(b * PWIN, PWIN)]],
                         g_v.at[b], sem)

    def _grp(o, carry):
        for b in range(GRP):
            w = o * GRP + b
            pltpu.make_async_copy(col3_hbm.at[tid, 0], idxc_v.at[b],
                                  sem_c).wait()
            pltpu.make_async_copy(h2_hbm.at[idxr_v.at[pl.ds(0, PWIN)]],
                                  g_v.at[b], sem).wait()
            pltpu.sync_copy(g_v.at[b], acc_sh.at[idxc_v.at[b]], add=True)

            @pl.when(o < NGRP - 1)
            def _():
                pltpu.async_copy(col3_hbm.at[tid, w + GRP], idxc_v.at[b],
                                 sem_c)
                pltpu.async_copy(
                    h2_hbm.at[idxr_v.at[pl.ds((w + GRP) * PWIN, PWIN)]],
                    g_v.at[b], sem)
        return carry

    lax.fori_loop(0, NGRP, _grp, 0)
    plsc.subcore_barrier()

    @pl.when(sid < NS - 1)
    def _():
        pltpu.sync_copy(acc_sh.at[pl.ds(rbase, ROWS_A)],
                        out_hbm.at[cid, pl.ds(rbase, ROWS_A)])

    @pl.when(sid == NS - 1)
    def _():
        pltpu.sync_copy(acc_sh.at[pl.ds(15 * ROWS_A, ROWS_B)],
                        out_hbm.at[cid, pl.ds(15 * ROWS_A, ROWS_B)])


def _h2_body(x_ref, w_ref, d0_ref, d1_ref, h2_ref):
    deg = d0_ref[...] + d1_ref[...] + 1.0
    dinv = lax.rsqrt(deg)
    h = jnp.dot(x_ref[...], w_ref[...], preferred_element_type=jnp.float32)
    h2_ref[...] = h * dinv


def _combine_body(p0_ref, p1_ref, h2_ref, d0_ref, d1_ref, b_ref, o_ref):
    deg = d0_ref[...] + d1_ref[...] + 1.0
    dinv = lax.rsqrt(deg)
    s = p0_ref[0] + p1_ref[0] - h2_ref[...]
    o_ref[...] = s * dinv + b_ref[...]


_BLK = 2000


def kernel(x, edge_index, kernel, bias):
    row = edge_index[0]
    col = edge_index[1]
    row3 = row.reshape(NC * NS, NWIN, WIN)
    col3 = col.reshape(NC * NS, PNWIN, PWIN)

    deg_part = _deg_kernel(row3)
    d0 = deg_part[:N].reshape(N, 1)
    d1 = deg_part[NPAD:NPAD + N].reshape(N, 1)

    grid = N // _BLK
    h2 = pl.pallas_call(
        _h2_body,
        grid=(grid,),
        in_specs=[
            pl.BlockSpec((_BLK, F), lambda i: (i, 0)),
            pl.BlockSpec((F, U), lambda i: (0, 0)),
            pl.BlockSpec((_BLK, 1), lambda i: (i, 0)),
            pl.BlockSpec((_BLK, 1), lambda i: (i, 0)),
        ],
        out_specs=pl.BlockSpec((_BLK, U), lambda i: (i, 0)),
        out_shape=jax.ShapeDtypeStruct((N, U), jnp.float32),
    )(x, kernel, d0, d1)

    p = _prop_kernel(h2, row, col3)

    out = pl.pallas_call(
        _combine_body,
        grid=(grid,),
        in_specs=[
            pl.BlockSpec((1, _BLK, U), lambda i: (0, i, 0)),
            pl.BlockSpec((1, _BLK, U), lambda i: (1, i, 0)),
            pl.BlockSpec((_BLK, U), lambda i: (i, 0)),
            pl.BlockSpec((_BLK, 1), lambda i: (i, 0)),
            pl.BlockSpec((_BLK, 1), lambda i: (i, 0)),
            pl.BlockSpec((1, U), lambda i: (0, 0)),
        ],
        out_specs=pl.BlockSpec((_BLK, U), lambda i: (i, 0)),
        out_shape=jax.ShapeDtypeStruct((N, U), jnp.float32),
    )(p, p, h2, d0, d1, bias.reshape(1, U))
    return out
